# baseline (speedup 1.0000x reference)
import functools

import numpy as np

import jax
import jax.numpy as jnp
from jax import lax
from jax.experimental import pallas as pl
from jax.experimental.pallas import tpu as pltpu
from jax.experimental.pallas import tpu_sc as plsc

N, C, H, W = 2, 256, 100, 152
R = 512
PH = PW = 7
SR = 2
SCALE = 0.125

NC, NS, L = 2, 16, 16
NW = NC * NS
OUT_ROWS = R * PH * PW
G = 16
GROUPS_PER_W = OUT_ROWS // (NW * G)
SLOTS = SR * SR * 4
GR = SLOTS * G

NB = 3
NI = 4
NO = 2

def _mesh():
    return plsc.VectorSubcoreMesh(
        core_axis_name="c", subcore_axis_name="s", num_cores=NC, num_subcores=NS
    )


@functools.partial(
    pl.kernel,
    out_type=jax.ShapeDtypeStruct((OUT_ROWS, C), jnp.float32),
    mesh=_mesh(),
    compiler_params=pltpu.CompilerParams(needs_layout_passes=False),
    scratch_types=[
        pltpu.VMEM((R * 5,), jnp.float32),
        pltpu.VMEM((NI * GR,), jnp.int32),
        pltpu.VMEM((NI * GR,), jnp.float32),
        pltpu.VMEM((NB * GR, C // 2), jnp.int32),
        pltpu.VMEM((NO * G, C), jnp.float32),
        pltpu.VMEM((NO, L), jnp.int32),
        pltpu.SemaphoreType.DMA,
        pltpu.SemaphoreType.DMA,
    ],
)
def _roi_pool_sc(feat_hbm, rois_hbm, out_hbm, rois_v, idx_v, w_v, buf_v,
                 ostage_v, oidx_v, sem_g, sem_o):
    wid = lax.axis_index("s") * NC + lax.axis_index("c")
    pltpu.sync_copy(rois_hbm, rois_v)

    def emit(g):
        si = lax.rem(g, NI) * GR
        sb = lax.rem(g, NB) * GR
        base = wid * (GROUPS_PER_W * G) + g * G
        lane16 = lax.iota(jnp.int32, L)
        orv = base + lane16
        r = lax.div(orv, PH * PW)
        rem = lax.rem(orv, PH * PW)
        ph = lax.div(rem, PW)
        pw = lax.rem(rem, PW)

        r5 = r * 5
        col = lambda c: plsc.load_gather(rois_v, [r5 + c])
        b_i = col(0).astype(jnp.int32)
        x1 = col(1) * SCALE - 0.5
        y1 = col(2) * SCALE - 0.5
        x2 = col(3) * SCALE - 0.5
        y2 = col(4) * SCALE - 0.5
        bw = jnp.maximum(x2 - x1, 1.0) * (1.0 / PW)
        bh = jnp.maximum(y2 - y1, 1.0) * (1.0 / PH)
        base_row = b_i * (H * W)
        ph_f = ph.astype(jnp.float32)
        pw_f = pw.astype(jnp.float32)

        wy, ry = [], []
        for s in range(SR):
            ys = y1 + (ph_f + (0.5 + s) / SR) * bh
            vy = jnp.where((ys > -1.0) & (ys < float(H)), 0.5, 0.0)
            yc = jnp.clip(ys, 0.0, float(H - 1))
            y0i = yc.astype(jnp.int32)
            ly = yc - y0i.astype(jnp.float32)
            wy.append([(1.0 - ly) * vy, ly * vy])
            ry.append([y0i * W, jnp.minimum(y0i + 1, H - 1) * W])
        wx, rx = [], []
        for t in range(SR):
            xs = x1 + (pw_f + (0.5 + t) / SR) * bw
            vx = jnp.where((xs > -1.0) & (xs < float(W)), 0.5, 0.0)
            xc = jnp.clip(xs, 0.0, float(W - 1))
            x0i = xc.astype(jnp.int32)
            lx = xc - x0i.astype(jnp.float32)
            wx.append([(1.0 - lx) * vx, lx * vx])
            rx.append([x0i, jnp.minimum(x0i + 1, W - 1)])

        k = 0
        for s in range(SR):
            for t in range(SR):
                for i in range(2):
                    for j in range(2):
                        idx_v[pl.ds(si + k * L, L)] = (
                            base_row + ry[s][i] + rx[t][j])
                        plsc.store_scatter(
                            w_v, [si + lane16 * SLOTS + k],
                            wy[s][i] * wx[t][j])
                        k += 1

        h = GR // 2
        pltpu.async_copy(feat_hbm.at[idx_v.at[pl.ds(si, h)]],
                         buf_v.at[pl.ds(sb, h)], sem_g)
        pltpu.async_copy(feat_hbm.at[idx_v.at[pl.ds(si + h, h)]],
                         buf_v.at[pl.ds(sb + h, h)], sem_g)

    for g0 in range(NB):
        emit(g0)

    def group_body(g, _):
        si = lax.rem(g, NI) * GR
        sb = lax.rem(g, NB) * GR
        soslot = lax.rem(g, NO)
        so = soslot * G
        base = wid * (GROUPS_PER_W * G) + g * G

        pltpu.make_async_copy(feat_hbm.at[pl.ds(0, GR)],
                              buf_v.at[pl.ds(sb, GR)], sem_g).wait()

        @pl.when(g >= NO)
        def _():
            pltpu.make_async_copy(out_hbm.at[pl.ds(0, G)],
                                  ostage_v.at[pl.ds(so, G)], sem_o).wait()

        def o_body(o, _):
            accs = [jnp.zeros((L,), jnp.float32) for _ in range(C // L)]
            wrow = w_v[pl.ds(si + o * SLOTS, SLOTS)]
            for kk in range(SLOTS):
                m = kk * L + o
                wv = wrow.at[lax.broadcast(kk, (L,))].get(
                    mode="promise_in_bounds")
                for j in range(C // 32):
                    a, b = plsc.unpack(
                        plsc.bitcast(buf_v[sb + m, pl.ds(j * L, L)],
                                     jnp.bfloat16),
                        format=plsc.PackFormat.INTERLEAVED,
                        preferred_element_type=jnp.float32,
                    )
                    accs[j] = accs[j] + wv * a
                    accs[j + C // 32] = accs[j + C // 32] + wv * b
            for j in range(C // 32):
                ostage_v[so + o, pl.ds(L * j, L)] = accs[j]
                ostage_v[so + o, pl.ds(C // 2 + L * j, L)] = (
                    accs[j + C // 32])
            return 0

        lax.fori_loop(0, G, o_body, 0)
        orv = base + lax.iota(jnp.int32, L)
        r = lax.div(orv, PH * PW)
        oidx_v[soslot] = lax.rem(orv, PH * PW) * R + r
        pltpu.async_copy(ostage_v.at[pl.ds(so, G)],
                         out_hbm.at[oidx_v.at[soslot]], sem_o)

        @pl.when(g + NB < GROUPS_PER_W)
        def _():
            emit(g + NB)

        return 0

    lax.fori_loop(0, GROUPS_PER_W, group_body, 0)
    for _ in range(NO):
        pltpu.make_async_copy(out_hbm.at[pl.ds(0, G)],
                              ostage_v.at[pl.ds(0, G)], sem_o).wait()


def kernel(input, rois):
    xb = input.astype(jnp.bfloat16)
    lo = lax.bitcast_convert_type(xb[:, :C // 2], jnp.uint16).astype(jnp.uint32)
    hi = lax.bitcast_convert_type(xb[:, C // 2:], jnp.uint16).astype(jnp.uint32)
    w = lo | (hi << 16)
    feat_i32 = lax.bitcast_convert_type(
        jnp.transpose(w, (0, 2, 3, 1)).reshape(N * H * W, C // 2), jnp.int32)
    out_rows = _roi_pool_sc(feat_i32, rois.reshape(-1))
    return out_rows.reshape(PH, PW, R, C).transpose(2, 3, 0, 1)


# speedup vs baseline: 1.0204x; 1.0204x over previous
import functools

import numpy as np

import jax
import jax.numpy as jnp
from jax import lax
from jax.experimental import pallas as pl
from jax.experimental.pallas import tpu as pltpu
from jax.experimental.pallas import tpu_sc as plsc

N, C, H, W = 2, 256, 100, 152
R = 512
PH = PW = 7
SR = 2
SCALE = 0.125

NC, NS, L = 2, 16, 16
NW = NC * NS
OUT_ROWS = R * PH * PW
G = 16
GROUPS_PER_W = OUT_ROWS // (NW * G)
SLOTS = SR * SR * 4
GR = SLOTS * G

NB = 3
NI = 4
NO = 2

def _mesh():
    return plsc.VectorSubcoreMesh(
        core_axis_name="c", subcore_axis_name="s", num_cores=NC, num_subcores=NS
    )


@functools.partial(
    pl.kernel,
    out_type=jax.ShapeDtypeStruct((OUT_ROWS, C), jnp.float32),
    mesh=_mesh(),
    compiler_params=pltpu.CompilerParams(needs_layout_passes=False),
    scratch_types=[
        pltpu.VMEM((R * 5,), jnp.float32),
        pltpu.VMEM((NI * GR,), jnp.int32),
        pltpu.VMEM((NI * GR,), jnp.float32),
        pltpu.VMEM((NB * GR, C // 2), jnp.int32),
        pltpu.VMEM((NO * G, C), jnp.float32),
        pltpu.VMEM((NO, L), jnp.int32),
        pltpu.SemaphoreType.DMA,
        pltpu.SemaphoreType.DMA,
    ],
)
def _roi_pool_sc(feat_hbm, rois_hbm, out_hbm, rois_v, idx_v, w_v, buf_v,
                 ostage_v, oidx_v, sem_g, sem_o):
    wid = lax.axis_index("s") * NC + lax.axis_index("c")
    pltpu.sync_copy(rois_hbm, rois_v)

    def emit(g):
        si = lax.rem(g, NI) * GR
        sb = lax.rem(g, NB) * GR
        base = wid * (GROUPS_PER_W * G) + g * G
        orv = base + lax.iota(jnp.int32, L)
        r = lax.div(orv, PH * PW)
        rem = lax.rem(orv, PH * PW)
        ph = lax.div(rem, PW)
        pw = lax.rem(rem, PW)

        r5 = r * 5
        col = lambda c: plsc.load_gather(rois_v, [r5 + c])
        b_i = col(0).astype(jnp.int32)
        x1 = col(1) * SCALE - 0.5
        y1 = col(2) * SCALE - 0.5
        x2 = col(3) * SCALE - 0.5
        y2 = col(4) * SCALE - 0.5
        bw = jnp.maximum(x2 - x1, 1.0) * (1.0 / PW)
        bh = jnp.maximum(y2 - y1, 1.0) * (1.0 / PH)
        base_row = b_i * (H * W)
        ph_f = ph.astype(jnp.float32)
        pw_f = pw.astype(jnp.float32)

        wy, ry = [], []
        for s in range(SR):
            ys = y1 + (ph_f + (0.5 + s) / SR) * bh
            vy = jnp.where((ys > -1.0) & (ys < float(H)), 0.5, 0.0)
            yc = jnp.clip(ys, 0.0, float(H - 1))
            y0i = yc.astype(jnp.int32)
            ly = yc - y0i.astype(jnp.float32)
            wy.append([(1.0 - ly) * vy, ly * vy])
            ry.append([y0i * W, jnp.minimum(y0i + 1, H - 1) * W])
        wx, rx = [], []
        for t in range(SR):
            xs = x1 + (pw_f + (0.5 + t) / SR) * bw
            vx = jnp.where((xs > -1.0) & (xs < float(W)), 0.5, 0.0)
            xc = jnp.clip(xs, 0.0, float(W - 1))
            x0i = xc.astype(jnp.int32)
            lx = xc - x0i.astype(jnp.float32)
            wx.append([(1.0 - lx) * vx, lx * vx])
            rx.append([x0i, jnp.minimum(x0i + 1, W - 1)])

        k = 0
        for s in range(SR):
            for t in range(SR):
                for i in range(2):
                    for j in range(2):
                        idx_v[pl.ds(si + k * L, L)] = (
                            base_row + ry[s][i] + rx[t][j])
                        w_v[pl.ds(si + k * L, L)] = wy[s][i] * wx[t][j]
                        k += 1

        h = GR // 2
        pltpu.async_copy(feat_hbm.at[idx_v.at[pl.ds(si, h)]],
                         buf_v.at[pl.ds(sb, h)], sem_g)
        pltpu.async_copy(feat_hbm.at[idx_v.at[pl.ds(si + h, h)]],
                         buf_v.at[pl.ds(sb + h, h)], sem_g)

    for g0 in range(NB):
        emit(g0)

    def group_body(g, _):
        si = lax.rem(g, NI) * GR
        sb = lax.rem(g, NB) * GR
        soslot = lax.rem(g, NO)
        so = soslot * G
        base = wid * (GROUPS_PER_W * G) + g * G

        pltpu.make_async_copy(feat_hbm.at[pl.ds(0, GR)],
                              buf_v.at[pl.ds(sb, GR)], sem_g).wait()

        @pl.when(g >= NO)
        def _():
            pltpu.make_async_copy(out_hbm.at[pl.ds(0, G)],
                                  ostage_v.at[pl.ds(so, G)], sem_o).wait()

        def o_body(o, _):
            accs = [jnp.zeros((L,), jnp.float32) for _ in range(C // L)]
            for kk in range(SLOTS):
                m = kk * L + o
                wv = plsc.load_gather(w_v, [lax.broadcast(si + m, (L,))])
                for j in range(C // 32):
                    a, b = plsc.unpack(
                        plsc.bitcast(buf_v[sb + m, pl.ds(j * L, L)],
                                     jnp.bfloat16),
                        format=plsc.PackFormat.INTERLEAVED,
                        preferred_element_type=jnp.float32,
                    )
                    accs[j] = accs[j] + wv * a
                    accs[j + C // 32] = accs[j + C // 32] + wv * b
            for j in range(C // 32):
                ostage_v[so + o, pl.ds(L * j, L)] = accs[j]
                ostage_v[so + o, pl.ds(C // 2 + L * j, L)] = (
                    accs[j + C // 32])
            return 0

        lax.fori_loop(0, G, o_body, 0)
        orv = base + lax.iota(jnp.int32, L)
        r = lax.div(orv, PH * PW)
        oidx_v[soslot] = lax.rem(orv, PH * PW) * R + r
        pltpu.async_copy(ostage_v.at[pl.ds(so, G)],
                         out_hbm.at[oidx_v.at[soslot]], sem_o)

        @pl.when(g + NB < GROUPS_PER_W)
        def _():
            emit(g + NB)

        return 0

    lax.fori_loop(0, GROUPS_PER_W, group_body, 0)
    for _ in range(NO):
        pltpu.make_async_copy(out_hbm.at[pl.ds(0, G)],
                              ostage_v.at[pl.ds(0, G)], sem_o).wait()


def kernel(input, rois):
    xb = input.astype(jnp.bfloat16)
    lo = lax.bitcast_convert_type(xb[:, :C // 2], jnp.uint16).astype(jnp.uint32)
    hi = lax.bitcast_convert_type(xb[:, C // 2:], jnp.uint16).astype(jnp.uint32)
    w = lo | (hi << 16)
    feat_i32 = lax.bitcast_convert_type(
        jnp.transpose(w, (0, 2, 3, 1)).reshape(N * H * W, C // 2), jnp.int32)
    out_rows = _roi_pool_sc(feat_i32, rois.reshape(-1))
    return out_rows.reshape(PH, PW, R, C).transpose(2, 3, 0, 1)
